# baseline (speedup 1.0000x reference)
import functools

import jax
import jax.numpy as jnp
from jax import lax
from jax.experimental import pallas as pl
from jax.experimental.pallas import tpu as pltpu
from jax.experimental.pallas import tpu_sc as plsc

B, N, C, K = 8, 10000, 128, 9
G = 4
EPS = 1e-5

NW = 32
CHUNK = 320
N_PAD = NW * CHUNK
NB = 8
NBLK = CHUNK // NB
V16 = C // 16
KA = 4


def _proj_body(x_ref, w_ref, b_ref, p_ref, xt_ref):
    xb = x_ref[0]
    p = jnp.dot(xb, w_ref[...], preferred_element_type=jnp.float32)
    p_ref[...] = (p + b_ref[0]).reshape(1, N, 1)
    xt_ref[...] = xb.astype(jnp.bfloat16)


def _project(x, ro_W, ro_b):
    return pl.pallas_call(
        _proj_body,
        grid=(B,),
        in_specs=[
            pl.BlockSpec((1, N, C), lambda b: (b, 0, 0)),
            pl.BlockSpec((C, 1), lambda b: (0, 0)),
            pl.BlockSpec((1,), lambda b: (0,)),
        ],
        out_specs=[
            pl.BlockSpec((1, N, 1), lambda b: (b, 0, 0)),
            pl.BlockSpec((N, C), lambda b: (0, b)),
        ],
        out_shape=[
            jax.ShapeDtypeStruct((B, N, 1), jnp.float32),
            jax.ShapeDtypeStruct((N, B * C), jnp.bfloat16),
        ],
    )(x, ro_W, ro_b)


def _sc_pool_body(xt_hbm, idx_hbm, p_hbm, out_hbm,
                  idxr, idxs, pv, wbuf, rowsA, rowsB, outb, semA, semB):
    wid = lax.axis_index("s") * 2 + lax.axis_index("c")
    node_base = jnp.minimum(wid * CHUNK, N - CHUNK)
    pltpu.sync_copy(idx_hbm.at[pl.ds(node_base, CHUNK)], idxr)

    lane = lax.iota(jnp.int32, 16)
    for k in range(K):
        kvec = jnp.full((16,), k, jnp.int32)

        def tr_grp(j, _):
            rows16 = j * 16 + lane
            idxs[k, pl.ds(j * 16, 16)] = plsc.load_gather(idxr, [rows16, kvec])
            return 0

        lax.fori_loop(0, CHUNK // 16, tr_grp, 0)

    def weights_batch(b, _):
        pltpu.sync_copy(p_hbm.at[b], pv)

        def weights_grp(j, _):
            sl = pl.ds(j * 16, 16)
            acc = plsc.load_gather(pv, [idxs[0, sl]])
            for k in range(1, K):
                acc = acc + plsc.load_gather(pv, [idxs[k, sl]])
            s = jnp.minimum(jnp.abs(acc * (1.0 / K)) * K, float(K - 1))
            for k in range(K):
                wbuf[b, k, sl] = jnp.clip(s - float(k) + 1.0, 0.0, 1.0)
            return 0

        lax.fori_loop(0, CHUNK // 16, weights_grp, 0)
        return 0

    lax.fori_loop(0, B, weights_batch, 0)

    def fire(grp, blk):
        ks = range(KA) if grp == 0 else range(KA, K)
        buf, s = (rowsA, semA) if grp == 0 else (rowsB, semB)
        for k in ks:
            pltpu.async_copy(xt_hbm.at[idxs.at[k, pl.ds(blk * NB, NB)]],
                             buf.at[k if grp == 0 else k - KA], s)

    def drain(grp, blk):
        ks = range(KA) if grp == 0 else range(KA, K)
        buf, s = (rowsA, semA) if grp == 0 else (rowsB, semB)
        for k in ks:
            pltpu.make_async_copy(
                xt_hbm.at[idxs.at[k, pl.ds(blk * NB, NB)]],
                buf.at[k if grp == 0 else k - KA], s).wait()

    himask = jnp.int32(-65536)

    def decode(buf, k, n, b, u):
        word = plsc.bitcast(buf[k, n, pl.ds(b * C + u * 32, 32)], jnp.int32)
        even = plsc.bitcast(jnp.left_shift(word, 16), jnp.float32)
        odd = plsc.bitcast(jnp.bitwise_and(word, himask), jnp.float32)
        return even, odd

    def accum(grp, blk):
        for n in range(NB):
            nloc = blk * NB + n

            def acc_batch(b, _):
                if grp == 0:
                    ws = [wbuf[b, k, pl.ds(nloc, 16)][0] for k in range(KA)]
                    for u in range(C // 32):
                        e0, o0 = decode(rowsA, 0, n, b, u)
                        ae, ao = ws[0] * e0, ws[0] * o0
                        for k in range(1, KA):
                            e, o = decode(rowsA, k, n, b, u)
                            ae, ao = ae + ws[k] * e, ao + ws[k] * o
                        outb[n, pl.ds(b * C + u * 32, 16)] = ae
                        outb[n, pl.ds(b * C + u * 32 + 16, 16)] = ao
                else:
                    ws = [wbuf[b, k, pl.ds(nloc, 16)][0]
                          for k in range(KA, K)]
                    for u in range(C // 32):
                        e0, o0 = decode(rowsB, 0, n, b, u)
                        ae, ao = ws[0] * e0, ws[0] * o0
                        for k in range(KA + 1, K):
                            e, o = decode(rowsB, k - KA, n, b, u)
                            ae, ao = ae + ws[k - KA] * e, ao + ws[k - KA] * o
                        plsc.addupdate(outb.at[n, pl.ds(b * C + u * 32, 16)],
                                       ae)
                        plsc.addupdate(
                            outb.at[n, pl.ds(b * C + u * 32 + 16, 16)], ao)
                return 0

            lax.fori_loop(0, B, acc_batch, 0)

    fire(0, 0)
    fire(1, 0)

    def blk_body(blk, _):
        drain(0, blk)
        accum(0, blk)

        @pl.when(blk + 1 < NBLK)
        def _():
            fire(0, blk + 1)

        drain(1, blk)
        accum(1, blk)

        @pl.when(blk + 1 < NBLK)
        def _():
            fire(1, blk + 1)

        pltpu.sync_copy(outb, out_hbm.at[pl.ds(node_base + blk * NB, NB)])
        return 0

    lax.fori_loop(0, NBLK, blk_body, 0)


def _sc_pool(xt, idx_w, p):
    mesh = plsc.VectorSubcoreMesh(core_axis_name="c", subcore_axis_name="s")
    kern = functools.partial(
        pl.kernel,
        mesh=mesh,
        compiler_params=pltpu.CompilerParams(
            needs_layout_passes=False, use_tc_tiling_on_sc=False),
        out_type=jax.ShapeDtypeStruct((N, B * C), jnp.float32),
        scratch_types=[
            pltpu.VMEM((CHUNK, K), jnp.int32),
            pltpu.VMEM((K, CHUNK), jnp.int32),
            pltpu.VMEM((N,), jnp.float32),
            pltpu.VMEM((B, K, CHUNK + 16), jnp.float32),
            pltpu.VMEM((KA, NB, B * C), jnp.bfloat16),
            pltpu.VMEM((K - KA, NB, B * C), jnp.bfloat16),
            pltpu.VMEM((NB, B * C), jnp.float32),
            pltpu.SemaphoreType.DMA,
            pltpu.SemaphoreType.DMA,
        ],
    )(_sc_pool_body)
    return kern(xt, idx_w, p)


def _gn_body(y_ref, g_ref, b_ref, o_ref):
    y = y_ref[...]
    s_ch = jnp.sum(y, axis=0, keepdims=True)
    q_ch = jnp.sum(y * y, axis=0, keepdims=True)
    gi = lax.broadcasted_iota(jnp.int32, (C, C), 0) // (C // G)
    gj = lax.broadcasted_iota(jnp.int32, (C, C), 1) // (C // G)
    M = jnp.where(gi == gj, 1.0 / ((C // G) * N), 0.0).astype(jnp.float32)
    mean_c = jnp.dot(s_ch, M, preferred_element_type=jnp.float32)
    ex2_c = jnp.dot(q_ch, M, preferred_element_type=jnp.float32)
    var_c = ex2_c - mean_c * mean_c
    rstd_c = lax.rsqrt(var_c + EPS)
    gam = g_ref[...].reshape(1, C)
    bet = b_ref[...].reshape(1, C)
    yn = (y - mean_c) * (rstd_c * gam) + bet
    mi = lax.broadcasted_iota(jnp.int32, (C, C), 0)
    cj = lax.broadcasted_iota(jnp.int32, (C, C), 1)
    chan = (mi // 32) * 32 + 2 * (mi % 16) + (mi % 32) // 16
    P = jnp.where(chan == cj, 1.0, 0.0).astype(jnp.float32)
    o_ref[...] = jnp.dot(yn, P,
                         preferred_element_type=jnp.float32).reshape(1, N, C)


def _group_norm(pool_t, gamma, beta):
    return pl.pallas_call(
        _gn_body,
        grid=(B,),
        in_specs=[
            pl.BlockSpec((N, C), lambda b: (0, b)),
            pl.BlockSpec((C,), lambda b: (0,)),
            pl.BlockSpec((C,), lambda b: (0,)),
        ],
        out_specs=pl.BlockSpec((1, N, C), lambda b: (b, 0, 0)),
        out_shape=jax.ShapeDtypeStruct((B, N, C), jnp.float32),
    )(pool_t, gamma, beta)


def kernel(x, dynamic_indices, ro_W, ro_b, gamma, beta):
    p3, xt = _project(x, ro_W, ro_b)
    p = p3.reshape(B, N)
    pool_t = _sc_pool(xt, dynamic_indices, p)
    m = jnp.arange(C)
    perm = (m // 32) * 32 + 2 * (m % 16) + (m % 32) // 16
    return _group_norm(pool_t, gamma[perm], beta[perm])


# speedup vs baseline: 1.3097x; 1.3097x over previous
import functools

import jax
import jax.numpy as jnp
from jax import lax
from jax.experimental import pallas as pl
from jax.experimental.pallas import tpu as pltpu
from jax.experimental.pallas import tpu_sc as plsc

B, N, C, K = 8, 10000, 128, 9
G = 4
EPS = 1e-5

NW = 32
CHUNK = 320
N_PAD = NW * CHUNK
NB = 8
NBLK = CHUNK // NB
V16 = C // 16
KA = 4


def _proj_body(x_ref, w_ref, b_ref, p_ref, xt_ref):
    xb = x_ref[0]
    p = jnp.dot(xb, w_ref[...], preferred_element_type=jnp.float32)
    p_ref[...] = (p + b_ref[0]).reshape(1, N, 1)
    xt_ref[...] = xb


def _project(x, ro_W, ro_b):
    return pl.pallas_call(
        _proj_body,
        grid=(B,),
        in_specs=[
            pl.BlockSpec((1, N, C), lambda b: (b, 0, 0)),
            pl.BlockSpec((C, 1), lambda b: (0, 0)),
            pl.BlockSpec((1,), lambda b: (0,)),
        ],
        out_specs=[
            pl.BlockSpec((1, N, 1), lambda b: (b, 0, 0)),
            pl.BlockSpec((N, C), lambda b: (0, b)),
        ],
        out_shape=[
            jax.ShapeDtypeStruct((B, N, 1), jnp.float32),
            jax.ShapeDtypeStruct((N, B * C), jnp.float32),
        ],
    )(x, ro_W, ro_b)


def _sc_pool_body(xt_hbm, idx_hbm, p_hbm, out_hbm,
                  idxr, idxs, pv, wbuf, rowsA, rowsB, outb, semA, semB,
                  semO):
    wid = lax.axis_index("s") * 2 + lax.axis_index("c")
    node_base = jnp.minimum(wid * CHUNK, N - CHUNK)
    pltpu.sync_copy(idx_hbm.at[pl.ds(node_base, CHUNK)], idxr)

    lane = lax.iota(jnp.int32, 16)
    for k in range(K):
        kvec = jnp.full((16,), k, jnp.int32)

        def tr_grp(j, _):
            rows16 = j * 16 + lane
            idxs[k, pl.ds(j * 16, 16)] = plsc.load_gather(idxr, [rows16, kvec])
            return 0

        lax.fori_loop(0, CHUNK // 16, tr_grp, 0)

    def weights_batch(b, _):
        pltpu.sync_copy(p_hbm.at[b], pv)

        def weights_grp(j, _):
            sl = pl.ds(j * 16, 16)
            acc = plsc.load_gather(pv, [idxs[0, sl]])
            for k in range(1, K):
                acc = acc + plsc.load_gather(pv, [idxs[k, sl]])
            s = jnp.minimum(jnp.abs(acc * (1.0 / K)) * K, float(K - 1))
            for k in range(K):
                wbuf[b, k, sl] = jnp.clip(s - float(k) + 1.0, 0.0, 1.0)
            return 0

        lax.fori_loop(0, CHUNK // 16, weights_grp, 0)
        return 0

    lax.fori_loop(0, B, weights_batch, 0)

    def fire(grp, blk):
        ks = range(KA) if grp == 0 else range(KA, K)
        buf, s = (rowsA, semA) if grp == 0 else (rowsB, semB)
        for k in ks:
            pltpu.async_copy(xt_hbm.at[idxs.at[k, pl.ds(blk * NB, NB)]],
                             buf.at[k if grp == 0 else k - KA], s)

    def drain(grp, blk):
        ks = range(KA) if grp == 0 else range(KA, K)
        buf, s = (rowsA, semA) if grp == 0 else (rowsB, semB)
        for k in ks:
            pltpu.make_async_copy(
                xt_hbm.at[idxs.at[k, pl.ds(blk * NB, NB)]],
                buf.at[k if grp == 0 else k - KA], s).wait()

    def accum(grp, blk):
        def acc_node(n, _):
            nloc = blk * NB + n
            for b in range(B):
                if grp == 0:
                    ws = [wbuf[b, k, pl.ds(nloc, 16)][0] for k in range(KA)]
                    for v in range(V16):
                        sl = pl.ds(b * C + v * 16, 16)
                        acc = ws[0] * rowsA[0, n, sl]
                        for k in range(1, KA):
                            acc = acc + ws[k] * rowsA[k, n, sl]
                        outb[n, sl] = acc
                else:
                    ws = [wbuf[b, k, pl.ds(nloc, 16)][0]
                          for k in range(KA, K)]
                    for v in range(V16):
                        sl = pl.ds(b * C + v * 16, 16)
                        acc = ws[0] * rowsB[0, n, sl]
                        for k in range(KA + 1, K):
                            acc = acc + ws[k - KA] * rowsB[k - KA, n, sl]
                        plsc.addupdate(outb.at[n, sl], acc)
            return 0

        lax.fori_loop(0, NB, acc_node, 0)

    fire(0, 0)
    fire(1, 0)

    def out_copy(blk):
        return pltpu.make_async_copy(
            outb, out_hbm.at[pl.ds(node_base + blk * NB, NB)], semO)

    def blk_body(blk, _):
        drain(0, blk)

        @pl.when(blk > 0)
        def _():
            out_copy(blk - 1).wait()

        accum(0, blk)

        @pl.when(blk + 1 < NBLK)
        def _():
            fire(0, blk + 1)

        drain(1, blk)
        accum(1, blk)

        @pl.when(blk + 1 < NBLK)
        def _():
            fire(1, blk + 1)

        out_copy(blk).start()
        return 0

    lax.fori_loop(0, NBLK, blk_body, 0)
    out_copy(NBLK - 1).wait()


def _sc_pool(xt, idx_w, p):
    mesh = plsc.VectorSubcoreMesh(core_axis_name="c", subcore_axis_name="s")
    kern = functools.partial(
        pl.kernel,
        mesh=mesh,
        compiler_params=pltpu.CompilerParams(
            needs_layout_passes=False, use_tc_tiling_on_sc=False),
        out_type=jax.ShapeDtypeStruct((N, B * C), jnp.float32),
        scratch_types=[
            pltpu.VMEM((CHUNK, K), jnp.int32),
            pltpu.VMEM((K, CHUNK), jnp.int32),
            pltpu.VMEM((N,), jnp.float32),
            pltpu.VMEM((B, K, CHUNK + 16), jnp.float32),
            pltpu.VMEM((KA, NB, B * C), jnp.float32),
            pltpu.VMEM((K - KA, NB, B * C), jnp.float32),
            pltpu.VMEM((NB, B * C), jnp.float32),
            pltpu.SemaphoreType.DMA,
            pltpu.SemaphoreType.DMA,
            pltpu.SemaphoreType.DMA,
        ],
    )(_sc_pool_body)
    return kern(xt, idx_w, p)


def _gn_body(y_ref, g_ref, b_ref, o_ref):
    y = y_ref[...]
    s_ch = jnp.sum(y, axis=0, keepdims=True)
    q_ch = jnp.sum(y * y, axis=0, keepdims=True)
    gi = lax.broadcasted_iota(jnp.int32, (C, C), 0) // (C // G)
    gj = lax.broadcasted_iota(jnp.int32, (C, C), 1) // (C // G)
    M = jnp.where(gi == gj, 1.0 / ((C // G) * N), 0.0).astype(jnp.float32)
    mean_c = jnp.dot(s_ch, M, preferred_element_type=jnp.float32)
    ex2_c = jnp.dot(q_ch, M, preferred_element_type=jnp.float32)
    var_c = ex2_c - mean_c * mean_c
    rstd_c = lax.rsqrt(var_c + EPS)
    gam = g_ref[...].reshape(1, C)
    bet = b_ref[...].reshape(1, C)
    o_ref[...] = ((y - mean_c) * (rstd_c * gam) + bet).reshape(1, N, C)


def _group_norm(pool_t, gamma, beta):
    return pl.pallas_call(
        _gn_body,
        grid=(B,),
        in_specs=[
            pl.BlockSpec((N, C), lambda b: (0, b)),
            pl.BlockSpec((C,), lambda b: (0,)),
            pl.BlockSpec((C,), lambda b: (0,)),
        ],
        out_specs=pl.BlockSpec((1, N, C), lambda b: (b, 0, 0)),
        out_shape=jax.ShapeDtypeStruct((B, N, C), jnp.float32),
    )(pool_t, gamma, beta)


def kernel(x, dynamic_indices, ro_W, ro_b, gamma, beta):
    p3, xt = _project(x, ro_W, ro_b)
    p = p3.reshape(B, N)
    pool_t = _sc_pool(xt, dynamic_indices, p)
    return _group_norm(pool_t, gamma, beta)


# speedup vs baseline: 1.4008x; 1.0695x over previous
import functools

import jax
import jax.numpy as jnp
from jax import lax
from jax.experimental import pallas as pl
from jax.experimental.pallas import tpu as pltpu
from jax.experimental.pallas import tpu_sc as plsc

B, N, C, K = 8, 10000, 128, 9
G = 4
EPS = 1e-5

NW = 32
CHUNK = 320
N_PAD = NW * CHUNK
NB = 8
NBLK = CHUNK // NB
V16 = C // 16
KA = 4


def _proj_body(x_ref, w_ref, b_ref, p_ref, xt_ref):
    xb = x_ref[0]
    p = jnp.dot(xb, w_ref[...], preferred_element_type=jnp.float32)
    p_ref[...] = (p + b_ref[0]).reshape(1, N, 1)
    xt_ref[...] = xb


def _project(x, ro_W, ro_b):
    return pl.pallas_call(
        _proj_body,
        grid=(B,),
        in_specs=[
            pl.BlockSpec((1, N, C), lambda b: (b, 0, 0)),
            pl.BlockSpec((C, 1), lambda b: (0, 0)),
            pl.BlockSpec((1,), lambda b: (0,)),
        ],
        out_specs=[
            pl.BlockSpec((1, N, 1), lambda b: (b, 0, 0)),
            pl.BlockSpec((N, C), lambda b: (0, b)),
        ],
        out_shape=[
            jax.ShapeDtypeStruct((B, N, 1), jnp.float32),
            jax.ShapeDtypeStruct((N, B * C), jnp.float32),
        ],
    )(x, ro_W, ro_b)


def _sc_pool_body(xt_hbm, idx_hbm, p_hbm, out_hbm,
                  idxr, idxs, pv, wbuf, smax, rowsA, rowsB, outb, semA,
                  semB, semO):
    wid = lax.axis_index("s") * 2 + lax.axis_index("c")
    node_base = jnp.minimum(wid * CHUNK, N - CHUNK)
    pltpu.sync_copy(idx_hbm.at[pl.ds(node_base, CHUNK)], idxr)

    lane = lax.iota(jnp.int32, 16)
    for k in range(K):
        kvec = jnp.full((16,), k, jnp.int32)

        def tr_grp(j, _):
            rows16 = j * 16 + lane
            idxs[k, pl.ds(j * 16, 16)] = plsc.load_gather(idxr, [rows16, kvec])
            return 0

        lax.fori_loop(0, CHUNK // 16, tr_grp, 0)

    zeros16 = jnp.zeros((16,), jnp.float32)

    def smax_init(j, _):
        smax[pl.ds(j * 16, 16)] = zeros16
        return 0

    lax.fori_loop(0, (CHUNK + 16) // 16, smax_init, 0)

    def weights_batch(b, _):
        pltpu.sync_copy(p_hbm.at[b], pv)

        def weights_grp(j, _):
            sl = pl.ds(j * 16, 16)
            acc = plsc.load_gather(pv, [idxs[0, sl]])
            for k in range(1, K):
                acc = acc + plsc.load_gather(pv, [idxs[k, sl]])
            s = jnp.minimum(jnp.abs(acc * (1.0 / K)) * K, float(K - 1))
            smax[sl] = jnp.maximum(smax[sl], s)
            for k in range(K):
                wbuf[b, k, sl] = jnp.clip(s - float(k) + 1.0, 0.0, 1.0)
            return 0

        lax.fori_loop(0, CHUNK // 16, weights_grp, 0)
        return 0

    lax.fori_loop(0, B, weights_batch, 0)

    def blk_smax(blk):
        return jnp.max(smax[pl.ds(blk * NB, 16)], axis=0)

    def fire(grp, blk):
        if grp == 0:
            for k in range(KA):
                pltpu.async_copy(xt_hbm.at[idxs.at[k, pl.ds(blk * NB, NB)]],
                                 rowsA.at[k], semA)
        else:
            sm = blk_smax(blk)
            for k in range(KA, K):
                def go(k=k):
                    pltpu.async_copy(
                        xt_hbm.at[idxs.at[k, pl.ds(blk * NB, NB)]],
                        rowsB.at[k - KA], semB)
                if k == KA:
                    go()
                else:
                    pl.when(sm > float(k - 1))(go)

    def drain(grp, blk):
        if grp == 0:
            for k in range(KA):
                pltpu.make_async_copy(
                    xt_hbm.at[idxs.at[k, pl.ds(blk * NB, NB)]],
                    rowsA.at[k], semA).wait()
        else:
            sm = blk_smax(blk)
            for k in range(KA, K):
                def go(k=k):
                    pltpu.make_async_copy(
                        xt_hbm.at[idxs.at[k, pl.ds(blk * NB, NB)]],
                        rowsB.at[k - KA], semB).wait()
                if k == KA:
                    go()
                else:
                    pl.when(sm > float(k - 1))(go)

    def accum(grp, blk):
        def acc_node(n, _):
            nloc = blk * NB + n
            for b in range(B):
                if grp == 0:
                    ws = [wbuf[b, k, pl.ds(nloc, 16)][0] for k in range(KA)]
                    for v in range(V16):
                        sl = pl.ds(b * C + v * 16, 16)
                        acc = ws[0] * rowsA[0, n, sl]
                        for k in range(1, KA):
                            acc = acc + ws[k] * rowsA[k, n, sl]
                        outb[n, sl] = acc
                else:
                    w0 = wbuf[b, KA, pl.ds(nloc, 16)][0]
                    for v in range(V16):
                        sl = pl.ds(b * C + v * 16, 16)
                        plsc.addupdate(outb.at[n, sl], w0 * rowsB[0, n, sl])
            return 0

        lax.fori_loop(0, NB, acc_node, 0)

        if grp == 1:
            sm = blk_smax(blk)
            for k in range(KA + 1, K):
                def go(k=k):
                    def acc_k(n, _):
                        nloc = blk * NB + n
                        for b in range(B):
                            w = wbuf[b, k, pl.ds(nloc, 16)][0]
                            for v in range(V16):
                                sl = pl.ds(b * C + v * 16, 16)
                                plsc.addupdate(outb.at[n, sl],
                                               w * rowsB[k - KA, n, sl])
                        return 0

                    lax.fori_loop(0, NB, acc_k, 0)

                pl.when(sm > float(k - 1))(go)

    fire(0, 0)
    fire(1, 0)

    def out_copy(blk):
        return pltpu.make_async_copy(
            outb, out_hbm.at[pl.ds(node_base + blk * NB, NB)], semO)

    def blk_body(blk, _):
        drain(0, blk)

        @pl.when(blk > 0)
        def _():
            out_copy(blk - 1).wait()

        accum(0, blk)

        @pl.when(blk + 1 < NBLK)
        def _():
            fire(0, blk + 1)

        drain(1, blk)
        accum(1, blk)

        @pl.when(blk + 1 < NBLK)
        def _():
            fire(1, blk + 1)

        out_copy(blk).start()
        return 0

    lax.fori_loop(0, NBLK, blk_body, 0)
    out_copy(NBLK - 1).wait()


def _sc_pool(xt, idx_w, p):
    mesh = plsc.VectorSubcoreMesh(core_axis_name="c", subcore_axis_name="s")
    kern = functools.partial(
        pl.kernel,
        mesh=mesh,
        compiler_params=pltpu.CompilerParams(
            needs_layout_passes=False, use_tc_tiling_on_sc=False),
        out_type=jax.ShapeDtypeStruct((N, B * C), jnp.float32),
        scratch_types=[
            pltpu.VMEM((CHUNK, K), jnp.int32),
            pltpu.VMEM((K, CHUNK), jnp.int32),
            pltpu.VMEM((N,), jnp.float32),
            pltpu.VMEM((B, K, CHUNK + 16), jnp.float32),
            pltpu.VMEM((CHUNK + 16,), jnp.float32),
            pltpu.VMEM((KA, NB, B * C), jnp.float32),
            pltpu.VMEM((K - KA, NB, B * C), jnp.float32),
            pltpu.VMEM((NB, B * C), jnp.float32),
            pltpu.SemaphoreType.DMA,
            pltpu.SemaphoreType.DMA,
            pltpu.SemaphoreType.DMA,
        ],
    )(_sc_pool_body)
    return kern(xt, idx_w, p)


def _gn_body(y_ref, g_ref, b_ref, o_ref):
    y = y_ref[...]
    s_ch = jnp.sum(y, axis=0, keepdims=True)
    q_ch = jnp.sum(y * y, axis=0, keepdims=True)
    gi = lax.broadcasted_iota(jnp.int32, (C, C), 0) // (C // G)
    gj = lax.broadcasted_iota(jnp.int32, (C, C), 1) // (C // G)
    M = jnp.where(gi == gj, 1.0 / ((C // G) * N), 0.0).astype(jnp.float32)
    mean_c = jnp.dot(s_ch, M, preferred_element_type=jnp.float32)
    ex2_c = jnp.dot(q_ch, M, preferred_element_type=jnp.float32)
    var_c = ex2_c - mean_c * mean_c
    rstd_c = lax.rsqrt(var_c + EPS)
    gam = g_ref[...].reshape(1, C)
    bet = b_ref[...].reshape(1, C)
    o_ref[...] = ((y - mean_c) * (rstd_c * gam) + bet).reshape(1, N, C)


def _group_norm(pool_t, gamma, beta):
    return pl.pallas_call(
        _gn_body,
        grid=(B,),
        in_specs=[
            pl.BlockSpec((N, C), lambda b: (0, b)),
            pl.BlockSpec((C,), lambda b: (0,)),
            pl.BlockSpec((C,), lambda b: (0,)),
        ],
        out_specs=pl.BlockSpec((1, N, C), lambda b: (b, 0, 0)),
        out_shape=jax.ShapeDtypeStruct((B, N, C), jnp.float32),
    )(pool_t, gamma, beta)


def kernel(x, dynamic_indices, ro_W, ro_b, gamma, beta):
    p3, xt = _project(x, ro_W, ro_b)
    p = p3.reshape(B, N)
    pool_t = _sc_pool(xt, dynamic_indices, p)
    return _group_norm(pool_t, gamma, beta)


# speedup vs baseline: 1.4473x; 1.0332x over previous
import functools

import jax
import jax.numpy as jnp
from jax import lax
from jax.experimental import pallas as pl
from jax.experimental.pallas import tpu as pltpu
from jax.experimental.pallas import tpu_sc as plsc

B, N, C, K = 8, 10000, 128, 9
G = 4
EPS = 1e-5

NW = 32
CHUNK = 320
N_PAD = NW * CHUNK
NB = 8
NBLK = CHUNK // NB
V16 = C // 16
TN = 400
KA = 4


def _proj_body(x_ref, w_ref, b_ref, p_ref, xt_ref):
    xb = x_ref[...]
    p = jnp.dot(xb.reshape(B * TN, C), w_ref[...],
                preferred_element_type=jnp.float32)
    p_ref[...] = p.reshape(B, TN, 1) + b_ref[0]
    for b in range(B):
        xt_ref[:, b, :] = xb[b]


def _project(x, ro_W, ro_b):
    return pl.pallas_call(
        _proj_body,
        grid=(N // TN,),
        in_specs=[
            pl.BlockSpec((B, TN, C), lambda t: (0, t, 0)),
            pl.BlockSpec((C, 1), lambda t: (0, 0)),
            pl.BlockSpec((1,), lambda t: (0,)),
        ],
        out_specs=[
            pl.BlockSpec((B, TN, 1), lambda t: (0, t, 0)),
            pl.BlockSpec((TN, B, C), lambda t: (t, 0, 0)),
        ],
        out_shape=[
            jax.ShapeDtypeStruct((B, N, 1), jnp.float32),
            jax.ShapeDtypeStruct((N, B, C), jnp.float32),
        ],
    )(x, ro_W, ro_b)


def _sc_pool_body(xt_hbm, idx_hbm, p_hbm, out_hbm,
                  idxr, idxs, pv, wbuf, smax, rowsA, rowsB, outb, semA,
                  semB, semO):
    wid = lax.axis_index("s") * 2 + lax.axis_index("c")
    node_base = jnp.minimum(wid * CHUNK, N - CHUNK)
    pltpu.sync_copy(idx_hbm.at[pl.ds(node_base, CHUNK)], idxr)

    lane = lax.iota(jnp.int32, 16)
    for k in range(K):
        kvec = jnp.full((16,), k, jnp.int32)

        def tr_grp(j, _):
            rows16 = j * 16 + lane
            idxs[k, pl.ds(j * 16, 16)] = plsc.load_gather(idxr, [rows16, kvec])
            return 0

        lax.fori_loop(0, CHUNK // 16, tr_grp, 0)

    zeros16 = jnp.zeros((16,), jnp.float32)

    def smax_init(j, _):
        smax[pl.ds(j * 16, 16)] = zeros16
        return 0

    lax.fori_loop(0, (CHUNK + 16) // 16, smax_init, 0)

    def weights_batch(b, _):
        pltpu.sync_copy(p_hbm.at[b], pv)

        def weights_grp(j, _):
            sl = pl.ds(j * 16, 16)
            acc = plsc.load_gather(pv, [idxs[0, sl]])
            for k in range(1, K):
                acc = acc + plsc.load_gather(pv, [idxs[k, sl]])
            s = jnp.minimum(jnp.abs(acc * (1.0 / K)) * K, float(K - 1))
            smax[sl] = jnp.maximum(smax[sl], s)
            for k in range(K):
                wbuf[b, k, sl] = jnp.clip(s - float(k) + 1.0, 0.0, 1.0)
            return 0

        lax.fori_loop(0, CHUNK // 16, weights_grp, 0)
        return 0

    lax.fori_loop(0, B, weights_batch, 0)

    def blk_smax(blk):
        return jnp.max(smax[pl.ds(blk * NB, 16)], axis=0)

    def fire(grp, blk):
        if grp == 0:
            for k in range(KA):
                pltpu.async_copy(xt_hbm.at[idxs.at[k, pl.ds(blk * NB, NB)]],
                                 rowsA.at[k], semA)
        else:
            sm = blk_smax(blk)
            for k in range(KA, K):
                def go(k=k):
                    pltpu.async_copy(
                        xt_hbm.at[idxs.at[k, pl.ds(blk * NB, NB)]],
                        rowsB.at[k - KA], semB)
                if k == KA:
                    go()
                else:
                    pl.when(sm > float(k - 1))(go)

    def drain(grp, blk):
        if grp == 0:
            for k in range(KA):
                pltpu.make_async_copy(
                    xt_hbm.at[idxs.at[k, pl.ds(blk * NB, NB)]],
                    rowsA.at[k], semA).wait()
        else:
            sm = blk_smax(blk)
            for k in range(KA, K):
                def go(k=k):
                    pltpu.make_async_copy(
                        xt_hbm.at[idxs.at[k, pl.ds(blk * NB, NB)]],
                        rowsB.at[k - KA], semB).wait()
                if k == KA:
                    go()
                else:
                    pl.when(sm > float(k - 1))(go)

    def accum(grp, blk):
        def acc_node(n, _):
            nloc = blk * NB + n
            for b in range(B):
                if grp == 0:
                    ws = [wbuf[b, k, pl.ds(nloc, 16)][0] for k in range(KA)]
                    for v in range(V16):
                        sl = pl.ds(b * C + v * 16, 16)
                        slr = pl.ds(v * 16, 16)
                        acc = ws[0] * rowsA[0, n, b, slr]
                        for k in range(1, KA):
                            acc = acc + ws[k] * rowsA[k, n, b, slr]
                        outb[n, sl] = acc
                else:
                    w0 = wbuf[b, KA, pl.ds(nloc, 16)][0]
                    for v in range(V16):
                        sl = pl.ds(b * C + v * 16, 16)
                        plsc.addupdate(outb.at[n, sl],
                                       w0 * rowsB[0, n, b, pl.ds(v * 16, 16)])
            return 0

        lax.fori_loop(0, NB, acc_node, 0)

        if grp == 1:
            sm = blk_smax(blk)
            for k in range(KA + 1, K):
                def go(k=k):
                    def acc_k(n, _):
                        nloc = blk * NB + n
                        for b in range(B):
                            w = wbuf[b, k, pl.ds(nloc, 16)][0]
                            for v in range(V16):
                                sl = pl.ds(b * C + v * 16, 16)
                                plsc.addupdate(
                                    outb.at[n, sl],
                                    w * rowsB[k - KA, n, b, pl.ds(v * 16, 16)])
                        return 0

                    lax.fori_loop(0, NB, acc_k, 0)

                pl.when(sm > float(k - 1))(go)

    fire(0, 0)
    fire(1, 0)

    def out_copy(blk):
        return pltpu.make_async_copy(
            outb, out_hbm.at[pl.ds(node_base + blk * NB, NB)], semO)

    def blk_body(blk, _):
        drain(0, blk)

        @pl.when(blk > 0)
        def _():
            out_copy(blk - 1).wait()

        accum(0, blk)

        @pl.when(blk + 1 < NBLK)
        def _():
            fire(0, blk + 1)

        drain(1, blk)
        accum(1, blk)

        @pl.when(blk + 1 < NBLK)
        def _():
            fire(1, blk + 1)

        out_copy(blk).start()
        return 0

    lax.fori_loop(0, NBLK, blk_body, 0)
    out_copy(NBLK - 1).wait()


def _sc_pool(xt, idx_w, p):
    mesh = plsc.VectorSubcoreMesh(core_axis_name="c", subcore_axis_name="s")
    kern = functools.partial(
        pl.kernel,
        mesh=mesh,
        compiler_params=pltpu.CompilerParams(
            needs_layout_passes=False, use_tc_tiling_on_sc=False),
        out_type=jax.ShapeDtypeStruct((N, B * C), jnp.float32),
        scratch_types=[
            pltpu.VMEM((CHUNK, K), jnp.int32),
            pltpu.VMEM((K, CHUNK), jnp.int32),
            pltpu.VMEM((N,), jnp.float32),
            pltpu.VMEM((B, K, CHUNK + 16), jnp.float32),
            pltpu.VMEM((CHUNK + 16,), jnp.float32),
            pltpu.VMEM((KA, NB, B, C), jnp.float32),
            pltpu.VMEM((K - KA, NB, B, C), jnp.float32),
            pltpu.VMEM((NB, B * C), jnp.float32),
            pltpu.SemaphoreType.DMA,
            pltpu.SemaphoreType.DMA,
            pltpu.SemaphoreType.DMA,
        ],
    )(_sc_pool_body)
    return kern(xt, idx_w, p)


def _gn_body(y_ref, g_ref, b_ref, o_ref):
    y = y_ref[...]
    s_ch = jnp.sum(y, axis=0, keepdims=True)
    q_ch = jnp.sum(y * y, axis=0, keepdims=True)
    gi = lax.broadcasted_iota(jnp.int32, (C, C), 0) // (C // G)
    gj = lax.broadcasted_iota(jnp.int32, (C, C), 1) // (C // G)
    M = jnp.where(gi == gj, 1.0 / ((C // G) * N), 0.0).astype(jnp.float32)
    mean_c = jnp.dot(s_ch, M, preferred_element_type=jnp.float32)
    ex2_c = jnp.dot(q_ch, M, preferred_element_type=jnp.float32)
    var_c = ex2_c - mean_c * mean_c
    rstd_c = lax.rsqrt(var_c + EPS)
    gam = g_ref[...].reshape(1, C)
    bet = b_ref[...].reshape(1, C)
    o_ref[...] = ((y - mean_c) * (rstd_c * gam) + bet).reshape(1, N, C)


def _group_norm(pool_t, gamma, beta):
    return pl.pallas_call(
        _gn_body,
        grid=(B,),
        in_specs=[
            pl.BlockSpec((N, C), lambda b: (0, b)),
            pl.BlockSpec((C,), lambda b: (0,)),
            pl.BlockSpec((C,), lambda b: (0,)),
        ],
        out_specs=pl.BlockSpec((1, N, C), lambda b: (b, 0, 0)),
        out_shape=jax.ShapeDtypeStruct((B, N, C), jnp.float32),
    )(pool_t, gamma, beta)


def kernel(x, dynamic_indices, ro_W, ro_b, gamma, beta):
    p3, xt = _project(x, ro_W, ro_b)
    p = p3.reshape(B, N)
    pool_t = _sc_pool(xt, dynamic_indices, p)
    return _group_norm(pool_t, gamma, beta)


# speedup vs baseline: 1.6284x; 1.1252x over previous
import functools

import jax
import jax.numpy as jnp
from jax import lax
from jax.experimental import pallas as pl
from jax.experimental.pallas import tpu as pltpu
from jax.experimental.pallas import tpu_sc as plsc

B, N, C, K = 8, 10000, 128, 9
G = 4
EPS = 1e-5

NW = 32
CHUNK = 320
N_PAD = NW * CHUNK
NB = 8
NBLK = CHUNK // NB
V16 = C // 16
TN = 400
KA = 4


def _proj_body(x_ref, w_ref, b_ref, p_ref, xt_ref):
    xb = x_ref[...]
    p = jnp.dot(xb.reshape(B * TN, C), w_ref[...],
                preferred_element_type=jnp.float32)
    p_ref[...] = p.reshape(B, TN, 1) + b_ref[0]
    for b in range(B):
        xt_ref[:, b, :] = xb[b]


def _project(x, ro_W, ro_b):
    return pl.pallas_call(
        _proj_body,
        grid=(N // TN,),
        in_specs=[
            pl.BlockSpec((B, TN, C), lambda t: (0, t, 0)),
            pl.BlockSpec((C, 1), lambda t: (0, 0)),
            pl.BlockSpec((1,), lambda t: (0,)),
        ],
        out_specs=[
            pl.BlockSpec((B, TN, 1), lambda t: (0, t, 0)),
            pl.BlockSpec((TN, B, C), lambda t: (t, 0, 0)),
        ],
        out_shape=[
            jax.ShapeDtypeStruct((B, N, 1), jnp.float32),
            jax.ShapeDtypeStruct((N, B, C), jnp.float32),
        ],
    )(x, ro_W, ro_b)


def _sc_pool_body(xt_hbm, idx_hbm, p_hbm, out_hbm,
                  idxr, idxs, pv, wbuf, smax, rowsA, rowsB, outb, semA,
                  semB, semO):
    wid = lax.axis_index("s") * 2 + lax.axis_index("c")
    node_base = jnp.minimum(wid * CHUNK, N - CHUNK)
    pltpu.sync_copy(idx_hbm.at[pl.ds(node_base, CHUNK)], idxr)

    lane = lax.iota(jnp.int32, 16)
    for k in range(K):
        kvec = jnp.full((16,), k, jnp.int32)

        def tr_grp(j, _):
            rows16 = j * 16 + lane
            idxs[k, pl.ds(j * 16, 16)] = plsc.load_gather(idxr, [rows16, kvec])
            return 0

        lax.fori_loop(0, CHUNK // 16, tr_grp, 0)

    zeros16 = jnp.zeros((16,), jnp.float32)

    def smax_init(j, _):
        smax[pl.ds(j * 16, 16)] = zeros16
        return 0

    lax.fori_loop(0, (CHUNK + 16) // 16, smax_init, 0)

    def weights_batch(b, _):
        pltpu.sync_copy(p_hbm.at[b], pv)

        def weights_grp(j, _):
            sl = pl.ds(j * 16, 16)
            acc = plsc.load_gather(pv, [idxs[0, sl]])
            for k in range(1, K):
                acc = acc + plsc.load_gather(pv, [idxs[k, sl]])
            s = jnp.minimum(jnp.abs(acc * (1.0 / K)) * K, float(K - 1))
            smax[sl] = jnp.maximum(smax[sl], s)
            for k in range(K):
                wbuf[b, k, sl] = jnp.clip(s - float(k) + 1.0, 0.0, 1.0)
            return 0

        lax.fori_loop(0, CHUNK // 16, weights_grp, 0)
        return 0

    lax.fori_loop(0, B, weights_batch, 0)

    def blk_smax(blk):
        return jnp.max(smax[pl.ds(blk * NB, 16)], axis=0)

    def fire(grp, blk):
        if grp == 0:
            for k in range(KA):
                pltpu.async_copy(xt_hbm.at[idxs.at[k, pl.ds(blk * NB, NB)]],
                                 rowsA.at[k], semA)
        else:
            sm = blk_smax(blk)
            for k in range(KA, K):
                def go(k=k):
                    pltpu.async_copy(
                        xt_hbm.at[idxs.at[k, pl.ds(blk * NB, NB)]],
                        rowsB.at[k - KA], semB)
                if k == KA:
                    go()
                else:
                    pl.when(sm > float(k - 1))(go)

    def drain(grp, blk):
        if grp == 0:
            for k in range(KA):
                pltpu.make_async_copy(
                    xt_hbm.at[idxs.at[k, pl.ds(blk * NB, NB)]],
                    rowsA.at[k], semA).wait()
        else:
            sm = blk_smax(blk)
            for k in range(KA, K):
                def go(k=k):
                    pltpu.make_async_copy(
                        xt_hbm.at[idxs.at[k, pl.ds(blk * NB, NB)]],
                        rowsB.at[k - KA], semB).wait()
                if k == KA:
                    go()
                else:
                    pl.when(sm > float(k - 1))(go)

    def accum(grp, blk):
        def acc_node(n, _):
            nloc = blk * NB + n
            for b in range(B):
                if grp == 0:
                    ws = [wbuf[b, k, pl.ds(nloc, 16)][0] for k in range(KA)]
                    for v in range(V16):
                        sl = pl.ds(b * C + v * 16, 16)
                        slr = pl.ds(v * 16, 16)
                        acc = ws[0] * rowsA[0, n, b, slr]
                        for k in range(1, KA):
                            acc = acc + ws[k] * rowsA[k, n, b, slr]
                        outb[n, sl] = acc
                else:
                    w0 = wbuf[b, KA, pl.ds(nloc, 16)][0]
                    for v in range(V16):
                        sl = pl.ds(b * C + v * 16, 16)
                        plsc.addupdate(outb.at[n, sl],
                                       w0 * rowsB[0, n, b, pl.ds(v * 16, 16)])
            return 0

        lax.fori_loop(0, NB, acc_node, 0)

        if grp == 1:
            sm = blk_smax(blk)
            for k in range(KA + 1, K):
                def go(k=k):
                    def acc_k(n, _):
                        nloc = blk * NB + n
                        for b in range(B):
                            w = wbuf[b, k, pl.ds(nloc, 16)][0]
                            for v in range(V16):
                                sl = pl.ds(b * C + v * 16, 16)
                                plsc.addupdate(
                                    outb.at[n, sl],
                                    w * rowsB[k - KA, n, b, pl.ds(v * 16, 16)])
                        return 0

                    lax.fori_loop(0, NB, acc_k, 0)

                pl.when(sm > float(k - 1))(go)

    fire(0, 0)
    fire(1, 0)

    def out_copies(blk):
        return [
            pltpu.make_async_copy(
                outb.at[:, pl.ds(b * C, C)],
                out_hbm.at[b, pl.ds(node_base + blk * NB, NB)], semO)
            for b in range(B)
        ]

    def blk_body(blk, _):
        drain(0, blk)

        @pl.when(blk > 0)
        def _():
            for cp in out_copies(blk - 1):
                cp.wait()

        accum(0, blk)

        @pl.when(blk + 1 < NBLK)
        def _():
            fire(0, blk + 1)

        drain(1, blk)
        accum(1, blk)

        @pl.when(blk + 1 < NBLK)
        def _():
            fire(1, blk + 1)

        for cp in out_copies(blk):
            cp.start()
        return 0

    lax.fori_loop(0, NBLK, blk_body, 0)
    for cp in out_copies(NBLK - 1):
        cp.wait()


def _sc_pool(xt, idx_w, p):
    mesh = plsc.VectorSubcoreMesh(core_axis_name="c", subcore_axis_name="s")
    kern = functools.partial(
        pl.kernel,
        mesh=mesh,
        compiler_params=pltpu.CompilerParams(
            needs_layout_passes=False, use_tc_tiling_on_sc=False),
        out_type=jax.ShapeDtypeStruct((B, N, C), jnp.float32),
        scratch_types=[
            pltpu.VMEM((CHUNK, K), jnp.int32),
            pltpu.VMEM((K, CHUNK), jnp.int32),
            pltpu.VMEM((N,), jnp.float32),
            pltpu.VMEM((B, K, CHUNK + 16), jnp.float32),
            pltpu.VMEM((CHUNK + 16,), jnp.float32),
            pltpu.VMEM((KA, NB, B, C), jnp.float32),
            pltpu.VMEM((K - KA, NB, B, C), jnp.float32),
            pltpu.VMEM((NB, B * C), jnp.float32),
            pltpu.SemaphoreType.DMA,
            pltpu.SemaphoreType.DMA,
            pltpu.SemaphoreType.DMA,
        ],
    )(_sc_pool_body)
    return kern(xt, idx_w, p)


def _gn_body(y_ref, g_ref, b_ref, o_ref):
    y = y_ref[0]
    s_ch = jnp.sum(y, axis=0, keepdims=True)
    q_ch = jnp.sum(y * y, axis=0, keepdims=True)
    gi = lax.broadcasted_iota(jnp.int32, (C, C), 0) // (C // G)
    gj = lax.broadcasted_iota(jnp.int32, (C, C), 1) // (C // G)
    M = jnp.where(gi == gj, 1.0 / ((C // G) * N), 0.0).astype(jnp.float32)
    mean_c = jnp.dot(s_ch, M, preferred_element_type=jnp.float32)
    ex2_c = jnp.dot(q_ch, M, preferred_element_type=jnp.float32)
    var_c = ex2_c - mean_c * mean_c
    rstd_c = lax.rsqrt(var_c + EPS)
    gam = g_ref[...].reshape(1, C)
    bet = b_ref[...].reshape(1, C)
    o_ref[...] = ((y - mean_c) * (rstd_c * gam) + bet).reshape(1, N, C)


def _group_norm(pool_t, gamma, beta):
    return pl.pallas_call(
        _gn_body,
        grid=(B,),
        in_specs=[
            pl.BlockSpec((1, N, C), lambda b: (b, 0, 0)),
            pl.BlockSpec((C,), lambda b: (0,)),
            pl.BlockSpec((C,), lambda b: (0,)),
        ],
        out_specs=pl.BlockSpec((1, N, C), lambda b: (b, 0, 0)),
        out_shape=jax.ShapeDtypeStruct((B, N, C), jnp.float32),
    )(pool_t, gamma, beta)


def kernel(x, dynamic_indices, ro_W, ro_b, gamma, beta):
    p3, xt = _project(x, ro_W, ro_b)
    p = p3.reshape(B, N)
    pool_t = _sc_pool(xt, dynamic_indices, p)
    return _group_norm(pool_t, gamma, beta)


# speedup vs baseline: 1.6328x; 1.0027x over previous
import functools

import jax
import jax.numpy as jnp
from jax import lax
from jax.experimental import pallas as pl
from jax.experimental.pallas import tpu as pltpu
from jax.experimental.pallas import tpu_sc as plsc

B, N, C, K = 8, 10000, 128, 9
G = 4
EPS = 1e-5

NW = 32
CHUNK = 320
N_PAD = NW * CHUNK
NB = 8
NBLK = CHUNK // NB
V16 = C // 16
TN = 400
KA = 4


def _proj_body(x_ref, w_ref, b_ref, p_ref, xt_ref):
    xb = x_ref[...]
    p = jnp.dot(xb.reshape(B * TN, C), w_ref[...],
                preferred_element_type=jnp.float32)
    p_ref[...] = p.reshape(B, TN, 1) + b_ref[0]
    for b in range(B):
        xt_ref[:, b, :] = xb[b]


def _project(x, ro_W, ro_b):
    return pl.pallas_call(
        _proj_body,
        grid=(N // TN,),
        in_specs=[
            pl.BlockSpec((B, TN, C), lambda t: (0, t, 0)),
            pl.BlockSpec((C, 1), lambda t: (0, 0)),
            pl.BlockSpec((1,), lambda t: (0,)),
        ],
        out_specs=[
            pl.BlockSpec((B, TN, 1), lambda t: (0, t, 0)),
            pl.BlockSpec((TN, B, C), lambda t: (t, 0, 0)),
        ],
        out_shape=[
            jax.ShapeDtypeStruct((B, N, 1), jnp.float32),
            jax.ShapeDtypeStruct((N, B, C), jnp.float32),
        ],
    )(x, ro_W, ro_b)


def _sc_pool_body(xt_hbm, idx_hbm, p_hbm, out_hbm,
                  idxr, idxs, pv, wbuf, smax, rowsA, rowsB, outb, semA,
                  semB, semO):
    wid = lax.axis_index("s") * 2 + lax.axis_index("c")
    node_base = jnp.minimum(wid * CHUNK, N - CHUNK)
    pltpu.sync_copy(idx_hbm.at[pl.ds(node_base, CHUNK)], idxr)

    lane = lax.iota(jnp.int32, 16)
    for k in range(K):
        kvec = jnp.full((16,), k, jnp.int32)

        def tr_grp(j, _):
            rows16 = j * 16 + lane
            idxs[k, pl.ds(j * 16, 16)] = plsc.load_gather(idxr, [rows16, kvec])
            return 0

        lax.fori_loop(0, CHUNK // 16, tr_grp, 0)

    zeros16 = jnp.zeros((16,), jnp.float32)

    def smax_init(j, _):
        smax[pl.ds(j * 16, 16)] = zeros16
        return 0

    lax.fori_loop(0, (CHUNK + 16) // 16, smax_init, 0)

    def weights_batch(b, _):
        pltpu.sync_copy(p_hbm.at[b], pv)

        def weights_grp(j, _):
            sl = pl.ds(j * 16, 16)
            acc = plsc.load_gather(pv, [idxs[0, sl]])
            for k in range(1, K):
                acc = acc + plsc.load_gather(pv, [idxs[k, sl]])
            s = jnp.minimum(jnp.abs(acc * (1.0 / K)) * K, float(K - 1))
            smax[sl] = jnp.maximum(smax[sl], s)
            for k in range(K):
                wbuf[b, k, sl] = jnp.clip(s - float(k) + 1.0, 0.0, 1.0)
            return 0

        lax.fori_loop(0, CHUNK // 16, weights_grp, 0)
        return 0


    def blk_smax(blk):
        return jnp.max(smax[pl.ds(blk * NB, 16)], axis=0)

    def fire(grp, blk):
        if grp == 0:
            for k in range(KA):
                pltpu.async_copy(xt_hbm.at[idxs.at[k, pl.ds(blk * NB, NB)]],
                                 rowsA.at[k], semA)
        else:
            sm = blk_smax(blk)
            for k in range(KA, K):
                def go(k=k):
                    pltpu.async_copy(
                        xt_hbm.at[idxs.at[k, pl.ds(blk * NB, NB)]],
                        rowsB.at[k - KA], semB)
                if k == KA:
                    go()
                else:
                    pl.when(jnp.logical_or(blk == 0, sm > float(k - 1)))(go)

    def drain(grp, blk):
        if grp == 0:
            for k in range(KA):
                pltpu.make_async_copy(
                    xt_hbm.at[idxs.at[k, pl.ds(blk * NB, NB)]],
                    rowsA.at[k], semA).wait()
        else:
            sm = blk_smax(blk)
            for k in range(KA, K):
                def go(k=k):
                    pltpu.make_async_copy(
                        xt_hbm.at[idxs.at[k, pl.ds(blk * NB, NB)]],
                        rowsB.at[k - KA], semB).wait()
                if k == KA:
                    go()
                else:
                    pl.when(jnp.logical_or(blk == 0, sm > float(k - 1)))(go)

    def accum(grp, blk):
        def acc_node(n, _):
            nloc = blk * NB + n
            for b in range(B):
                if grp == 0:
                    ws = [wbuf[b, k, pl.ds(nloc, 16)][0] for k in range(KA)]
                    for v in range(V16):
                        sl = pl.ds(b * C + v * 16, 16)
                        slr = pl.ds(v * 16, 16)
                        acc = ws[0] * rowsA[0, n, b, slr]
                        for k in range(1, KA):
                            acc = acc + ws[k] * rowsA[k, n, b, slr]
                        outb[n, sl] = acc
                else:
                    w0 = wbuf[b, KA, pl.ds(nloc, 16)][0]
                    for v in range(V16):
                        sl = pl.ds(b * C + v * 16, 16)
                        plsc.addupdate(outb.at[n, sl],
                                       w0 * rowsB[0, n, b, pl.ds(v * 16, 16)])
            return 0

        lax.fori_loop(0, NB, acc_node, 0)

        if grp == 1:
            sm = blk_smax(blk)
            for k in range(KA + 1, K):
                def go(k=k):
                    def acc_k(n, _):
                        nloc = blk * NB + n
                        for b in range(B):
                            w = wbuf[b, k, pl.ds(nloc, 16)][0]
                            for v in range(V16):
                                sl = pl.ds(b * C + v * 16, 16)
                                plsc.addupdate(
                                    outb.at[n, sl],
                                    w * rowsB[k - KA, n, b, pl.ds(v * 16, 16)])
                        return 0

                    lax.fori_loop(0, NB, acc_k, 0)

                pl.when(sm > float(k - 1))(go)

    def out_copies(blk):
        return [
            pltpu.make_async_copy(
                outb.at[:, pl.ds(b * C, C)],
                out_hbm.at[b, pl.ds(node_base + blk * NB, NB)], semO)
            for b in range(B)
        ]

    fire(0, 0)
    fire(1, 0)
    lax.fori_loop(0, B, weights_batch, 0)

    def blk_body(blk, _):
        drain(0, blk)

        @pl.when(blk > 0)
        def _():
            for cp in out_copies(blk - 1):
                cp.wait()

        accum(0, blk)

        @pl.when(blk + 1 < NBLK)
        def _():
            fire(0, blk + 1)

        drain(1, blk)
        accum(1, blk)

        @pl.when(blk + 1 < NBLK)
        def _():
            fire(1, blk + 1)

        for cp in out_copies(blk):
            cp.start()
        return 0

    lax.fori_loop(0, NBLK, blk_body, 0)
    for cp in out_copies(NBLK - 1):
        cp.wait()


def _sc_pool(xt, idx_w, p):
    mesh = plsc.VectorSubcoreMesh(core_axis_name="c", subcore_axis_name="s")
    kern = functools.partial(
        pl.kernel,
        mesh=mesh,
        compiler_params=pltpu.CompilerParams(
            needs_layout_passes=False, use_tc_tiling_on_sc=False),
        out_type=jax.ShapeDtypeStruct((B, N, C), jnp.float32),
        scratch_types=[
            pltpu.VMEM((CHUNK, K), jnp.int32),
            pltpu.VMEM((K, CHUNK), jnp.int32),
            pltpu.VMEM((N,), jnp.float32),
            pltpu.VMEM((B, K, CHUNK + 16), jnp.float32),
            pltpu.VMEM((CHUNK + 16,), jnp.float32),
            pltpu.VMEM((KA, NB, B, C), jnp.float32),
            pltpu.VMEM((K - KA, NB, B, C), jnp.float32),
            pltpu.VMEM((NB, B * C), jnp.float32),
            pltpu.SemaphoreType.DMA,
            pltpu.SemaphoreType.DMA,
            pltpu.SemaphoreType.DMA,
        ],
    )(_sc_pool_body)
    return kern(xt, idx_w, p)


def _gn_body(y_ref, g_ref, b_ref, o_ref):
    y = y_ref[0]
    s_ch = jnp.sum(y, axis=0, keepdims=True)
    q_ch = jnp.sum(y * y, axis=0, keepdims=True)
    gi = lax.broadcasted_iota(jnp.int32, (C, C), 0) // (C // G)
    gj = lax.broadcasted_iota(jnp.int32, (C, C), 1) // (C // G)
    M = jnp.where(gi == gj, 1.0 / ((C // G) * N), 0.0).astype(jnp.float32)
    mean_c = jnp.dot(s_ch, M, preferred_element_type=jnp.float32)
    ex2_c = jnp.dot(q_ch, M, preferred_element_type=jnp.float32)
    var_c = ex2_c - mean_c * mean_c
    rstd_c = lax.rsqrt(var_c + EPS)
    gam = g_ref[...].reshape(1, C)
    bet = b_ref[...].reshape(1, C)
    o_ref[...] = ((y - mean_c) * (rstd_c * gam) + bet).reshape(1, N, C)


def _group_norm(pool_t, gamma, beta):
    return pl.pallas_call(
        _gn_body,
        grid=(B,),
        in_specs=[
            pl.BlockSpec((1, N, C), lambda b: (b, 0, 0)),
            pl.BlockSpec((C,), lambda b: (0,)),
            pl.BlockSpec((C,), lambda b: (0,)),
        ],
        out_specs=pl.BlockSpec((1, N, C), lambda b: (b, 0, 0)),
        out_shape=jax.ShapeDtypeStruct((B, N, C), jnp.float32),
    )(pool_t, gamma, beta)


def kernel(x, dynamic_indices, ro_W, ro_b, gamma, beta):
    p3, xt = _project(x, ro_W, ro_b)
    p = p3.reshape(B, N)
    pool_t = _sc_pool(xt, dynamic_indices, p)
    return _group_norm(pool_t, gamma, beta)
